# baseline (speedup 1.0000x reference)
import jax
import jax.numpy as jnp
from jax.experimental import pallas as pl
from jax.experimental.pallas import tpu as pltpu

_BQ = 256
_NCHUNK = 4
_LOG2E = 1.4426950408889634


def _retrieve_kernel(q_ref, k_ref, w_ref, o_ref, kbf_ref, wbf_ref):
    i = pl.program_id(0)

    @pl.when(i == 0)
    def _():
        k = k_ref[...]
        kinv = 1.0 / (jnp.sqrt(jnp.sum(k * k, axis=1, keepdims=True)) + 1e-8)
        kbf_ref[...] = (k * kinv).astype(jnp.bfloat16)
        w = w_ref[...]
        wbf_ref[...] = jnp.concatenate(
            [w, jnp.ones((w.shape[0], 128), jnp.float32)], axis=1
        ).astype(jnp.bfloat16)

    q = q_ref[...]
    qn = q * (1.0 / (jnp.sqrt(jnp.sum(q * q, axis=1, keepdims=True)) + 1e-8))
    h = q.shape[1]
    sim = jax.lax.dot_general(
        qn.astype(jnp.bfloat16), kbf_ref[...], (((1,), (1,)), ((), ())),
        preferred_element_type=jnp.float32,
    )
    e = jnp.exp(sim)
    acc = jnp.dot(e.astype(jnp.bfloat16), wbf_ref[...],
                  preferred_element_type=jnp.float32)
    num = acc[:, :h]
    deninv = 1.0 / acc[:, h:]
    o_ref[...] = num * jnp.concatenate([deninv, deninv], axis=1)


def kernel(queries, keys, weights):
    nq, h = queries.shape
    ns = keys.shape[0]
    return pl.pallas_call(
        _retrieve_kernel,
        grid=(nq // _BQ,),
        in_specs=[
            pl.BlockSpec((_BQ, h), lambda i: (i, 0)),
            pl.BlockSpec((ns, h), lambda i: (0, 0)),
            pl.BlockSpec((ns, h), lambda i: (0, 0)),
        ],
        out_specs=pl.BlockSpec((_BQ, h), lambda i: (i, 0)),
        out_shape=jax.ShapeDtypeStruct((nq, h), jnp.float32),
        scratch_shapes=[
            pltpu.VMEM((ns, h), jnp.bfloat16),
            pltpu.VMEM((ns, h + 128), jnp.bfloat16),
        ],
    )(queries, keys, weights)


# speedup vs baseline: 1.3184x; 1.3184x over previous
import jax
import jax.numpy as jnp
from jax.experimental import pallas as pl
from jax.experimental.pallas import tpu as pltpu

_BQ = 256
_NCHUNK = 4
_LOG2E = 1.4426950408889634


def _retrieve_kernel(q_ref, k_ref, w_ref, o_ref, kbf_ref, wbf_ref):
    i = pl.program_id(0)

    @pl.when(i == 0)
    def _():
        k = k_ref[...]
        kinv = 1.0 / (jnp.sqrt(jnp.sum(k * k, axis=1, keepdims=True)) + 1e-8)
        kbf_ref[...] = (k * kinv).astype(jnp.bfloat16)
        wbf_ref[...] = w_ref[...].astype(jnp.bfloat16)

    q = q_ref[...]
    qn = q * (1.0 / (jnp.sqrt(jnp.sum(q * q, axis=1, keepdims=True)) + 1e-8))
    sim = jax.lax.dot_general(
        qn.astype(jnp.bfloat16), kbf_ref[...], (((1,), (1,)), ((), ())),
        preferred_element_type=jnp.float32,
    )
    e = jnp.exp(sim)
    den = jnp.sum(e, axis=1, keepdims=True)
    acc = jnp.dot(e.astype(jnp.bfloat16), wbf_ref[...],
                  preferred_element_type=jnp.float32)
    o_ref[...] = acc / den


def kernel(queries, keys, weights):
    nq, h = queries.shape
    ns = keys.shape[0]
    return pl.pallas_call(
        _retrieve_kernel,
        grid=(nq // _BQ,),
        in_specs=[
            pl.BlockSpec((_BQ, h), lambda i: (i, 0)),
            pl.BlockSpec((ns, h), lambda i: (0, 0)),
            pl.BlockSpec((ns, h), lambda i: (0, 0)),
        ],
        out_specs=pl.BlockSpec((_BQ, h), lambda i: (i, 0)),
        out_shape=jax.ShapeDtypeStruct((nq, h), jnp.float32),
        scratch_shapes=[
            pltpu.VMEM((ns, h), jnp.bfloat16),
            pltpu.VMEM((ns, h), jnp.bfloat16),
        ],
    )(queries, keys, weights)


# speedup vs baseline: 1.3198x; 1.0011x over previous
import jax
import jax.numpy as jnp
from jax.experimental import pallas as pl
from jax.experimental.pallas import tpu as pltpu

_BQ = 256
_NCHUNK = 4
_LOG2E = 1.4426950408889634


def _retrieve_kernel(q_ref, k_ref, w_ref, o_ref, kbf_ref, wbf_ref):
    i = pl.program_id(0)

    @pl.when(i == 0)
    def _():
        k = k_ref[...]
        kinv = 1.0 / (jnp.sqrt(jnp.sum(k * k, axis=1, keepdims=True)) + 1e-8)
        kbf_ref[...] = (k * kinv).astype(jnp.bfloat16)
        w = w_ref[...]
        wbf_ref[...] = jnp.concatenate(
            [w, jnp.ones((w.shape[0], 128), jnp.float32)], axis=1
        ).astype(jnp.bfloat16)

    q = q_ref[...]
    qn = q * (1.0 / (jnp.sqrt(jnp.sum(q * q, axis=1, keepdims=True)) + 1e-8))
    sim = jax.lax.dot_general(
        qn.astype(jnp.bfloat16), kbf_ref[...], (((1,), (1,)), ((), ())),
        preferred_element_type=jnp.float32,
    )
    e = jnp.exp(sim.astype(jnp.bfloat16))
    acc = jnp.dot(e, wbf_ref[...], preferred_element_type=jnp.float32)
    h = q.shape[1]
    deninv = 1.0 / acc[:, h:]
    o_ref[...] = acc[:, :h] * jnp.concatenate([deninv, deninv], axis=1)


def kernel(queries, keys, weights):
    nq, h = queries.shape
    ns = keys.shape[0]
    return pl.pallas_call(
        _retrieve_kernel,
        grid=(nq // _BQ,),
        in_specs=[
            pl.BlockSpec((_BQ, h), lambda i: (i, 0)),
            pl.BlockSpec((ns, h), lambda i: (0, 0)),
            pl.BlockSpec((ns, h), lambda i: (0, 0)),
        ],
        out_specs=pl.BlockSpec((_BQ, h), lambda i: (i, 0)),
        out_shape=jax.ShapeDtypeStruct((nq, h), jnp.float32),
        scratch_shapes=[
            pltpu.VMEM((ns, h), jnp.bfloat16),
            pltpu.VMEM((ns, h + 128), jnp.bfloat16),
        ],
    )(queries, keys, weights)
